# Initial kernel scaffold; baseline (speedup 1.0000x reference)
import functools

import jax
import jax.numpy as jnp
from jax import lax
from jax.experimental import pallas as pl
from jax.experimental.pallas import tpu as pltpu
from jax.experimental.pallas import tpu_sc as plsc

V = 100000
D = 64
B = 1024
CTX = 200

NC = 2
NS = 16
NW = NC * NS
RPW = B // NW
GCH = CTX // 2

VT = 512
NV = (V + VT - 1) // VT


def _gather_mean(inputs, table):
    mesh = plsc.VectorSubcoreMesh(
        core_axis_name="c", subcore_axis_name="s",
        num_cores=NC, num_subcores=NS)

    @functools.partial(
        pl.kernel,
        out_type=jax.ShapeDtypeStruct((B, D), jnp.float32),
        mesh=mesh,
        scratch_types=[
            pltpu.VMEM((RPW, CTX), jnp.int32),
            pltpu.VMEM((CTX, D), jnp.float32),
            pltpu.VMEM((RPW, D), jnp.float32),
            pltpu.SemaphoreType.DMA,
        ],
    )
    def body(inputs_hbm, table_hbm, out_hbm, idx_v, rows_v, out_v, sem):
        wid = lax.axis_index("s") * NC + lax.axis_index("c")
        base = wid * RPW
        pltpu.sync_copy(inputs_hbm.at[pl.ds(base, RPW)], idx_v)

        def row_body(e, _):
            cp0 = pltpu.async_copy(
                table_hbm.at[idx_v.at[e, pl.ds(0, GCH)]],
                rows_v.at[pl.ds(0, GCH)], sem)
            cp1 = pltpu.async_copy(
                table_hbm.at[idx_v.at[e, pl.ds(GCH, GCH)]],
                rows_v.at[pl.ds(GCH, GCH)], sem)
            cp0.wait()
            cp1.wait()

            def acc_body(r, accs):
                return tuple(accs[g] + rows_v[r, pl.ds(g * 16, 16)]
                             for g in range(D // 16))

            accs = lax.fori_loop(
                0, CTX, acc_body,
                tuple(jnp.zeros((16,), jnp.float32) for _ in range(D // 16)))
            for g in range(D // 16):
                out_v[e, pl.ds(g * 16, 16)] = accs[g] * (1.0 / CTX)
            return 0

        lax.fori_loop(0, RPW, row_body, 0)
        pltpu.sync_copy(out_v, out_hbm.at[pl.ds(base, RPW)])

    return body(inputs, table)


def _sumexp_body(avg_ref, w_ref, b_ref, s_ref):
    j = pl.program_id(0)

    @pl.when(j == 0)
    def _():
        s_ref[...] = jnp.zeros_like(s_ref)

    logits = jnp.dot(avg_ref[...], w_ref[...],
                     preferred_element_type=jnp.float32) + b_ref[...]
    e = jnp.exp(logits)
    col = j * VT + lax.broadcasted_iota(jnp.int32, (1, VT), 1)
    e = jnp.where(col < V, e, 0.0)
    s_ref[...] += e.reshape(B, VT // 128, 128).sum(axis=1)


def _normalize_body(avg_ref, w_ref, b_ref, s_ref, o_ref):
    logits = jnp.dot(avg_ref[...], w_ref[...],
                     preferred_element_type=jnp.float32) + b_ref[...]
    e = jnp.exp(logits)
    inv = 1.0 / jnp.sum(s_ref[...], axis=1, keepdims=True)
    o_ref[...] = e * inv


def kernel(inputs, table, W, b):
    avg = _gather_mean(inputs, table)
    b2 = b.reshape(1, V)

    s128 = pl.pallas_call(
        _sumexp_body,
        grid=(NV,),
        in_specs=[
            pl.BlockSpec((B, D), lambda j: (0, 0)),
            pl.BlockSpec((D, VT), lambda j: (0, j)),
            pl.BlockSpec((1, VT), lambda j: (0, j)),
        ],
        out_specs=pl.BlockSpec((B, 128), lambda j: (0, 0)),
        out_shape=jax.ShapeDtypeStruct((B, 128), jnp.float32),
    )(avg, W, b2)

    out = pl.pallas_call(
        _normalize_body,
        grid=(NV,),
        in_specs=[
            pl.BlockSpec((B, D), lambda j: (0, 0)),
            pl.BlockSpec((D, VT), lambda j: (0, j)),
            pl.BlockSpec((1, VT), lambda j: (0, j)),
            pl.BlockSpec((B, 128), lambda j: (0, 0)),
        ],
        out_specs=pl.BlockSpec((B, VT), lambda j: (0, j)),
        out_shape=jax.ShapeDtypeStruct((B, V), jnp.float32),
    )(avg, W, b2, s128)
    return out


# speedup vs baseline: 1.1655x; 1.1655x over previous
import functools

import jax
import jax.numpy as jnp
from jax import lax
from jax.experimental import pallas as pl
from jax.experimental.pallas import tpu as pltpu
from jax.experimental.pallas import tpu_sc as plsc

V = 100000
D = 64
B = 1024
CTX = 200

NC = 2
NS = 16
NW = NC * NS
RPW = B // NW
GC0 = 104
GC1 = CTX - GC0

VT = 512
NV = (V + VT - 1) // VT


def _gather_mean(inputs, table):
    mesh = plsc.VectorSubcoreMesh(
        core_axis_name="c", subcore_axis_name="s",
        num_cores=NC, num_subcores=NS)

    @functools.partial(
        pl.kernel,
        out_type=jax.ShapeDtypeStruct((B, D), jnp.float32),
        mesh=mesh,
        scratch_types=[
            pltpu.VMEM((RPW * CTX,), jnp.int32),
            pltpu.VMEM((CTX, D), jnp.float32),
            pltpu.VMEM((RPW, D), jnp.float32),
            pltpu.SemaphoreType.DMA,
        ],
        compiler_params=pltpu.CompilerParams(use_tc_tiling_on_sc=False),
    )
    def body(inputs_hbm, table_hbm, out_hbm, idx_v, rows_v, out_v, sem):
        wid = lax.axis_index("s") * NC + lax.axis_index("c")
        base = wid * RPW
        pltpu.sync_copy(inputs_hbm.at[pl.ds(base * CTX, RPW * CTX)], idx_v)

        def row_body(e, _):
            off = pl.multiple_of(e * CTX, 8)
            cp0 = pltpu.async_copy(
                table_hbm.at[idx_v.at[pl.ds(off, GC0)]],
                rows_v.at[pl.ds(0, GC0)], sem)
            cp1 = pltpu.async_copy(
                table_hbm.at[idx_v.at[pl.ds(pl.multiple_of(off + GC0, 8), GC1)]],
                rows_v.at[pl.ds(GC0, GC1)], sem)
            cp0.wait()
            cp1.wait()

            def acc_body(r, accs):
                return tuple(accs[g] + rows_v[r, pl.ds(g * 16, 16)]
                             for g in range(D // 16))

            accs = lax.fori_loop(
                0, CTX, acc_body,
                tuple(jnp.zeros((16,), jnp.float32) for _ in range(D // 16)))
            for g in range(D // 16):
                out_v[e, pl.ds(g * 16, 16)] = accs[g] * (1.0 / CTX)
            return 0

        lax.fori_loop(0, RPW, row_body, 0)
        pltpu.sync_copy(out_v, out_hbm.at[pl.ds(base, RPW)])

    return body(inputs.reshape(B * CTX), table)


def _sumexp_body(avg_ref, w_ref, b_ref, s_ref):
    j = pl.program_id(0)

    @pl.when(j == 0)
    def _():
        s_ref[...] = jnp.zeros_like(s_ref)

    logits = jnp.dot(avg_ref[...], w_ref[...],
                     preferred_element_type=jnp.float32) + b_ref[...]
    e = jnp.exp(logits)
    col = j * VT + lax.broadcasted_iota(jnp.int32, (1, VT), 1)
    e = jnp.where(col < V, e, 0.0)
    s_ref[...] += e.reshape(B, VT // 128, 128).sum(axis=1)


def _normalize_body(avg_ref, w_ref, b_ref, s_ref, o_ref):
    logits = jnp.dot(avg_ref[...], w_ref[...],
                     preferred_element_type=jnp.float32) + b_ref[...]
    e = jnp.exp(logits)
    inv = 1.0 / jnp.sum(s_ref[...], axis=1, keepdims=True)
    o_ref[...] = e * inv


def kernel(inputs, table, W, b):
    avg = _gather_mean(inputs, table)
    b2 = b.reshape(1, V)

    s128 = pl.pallas_call(
        _sumexp_body,
        grid=(NV,),
        in_specs=[
            pl.BlockSpec((B, D), lambda j: (0, 0)),
            pl.BlockSpec((D, VT), lambda j: (0, j)),
            pl.BlockSpec((1, VT), lambda j: (0, j)),
        ],
        out_specs=pl.BlockSpec((B, 128), lambda j: (0, 0)),
        out_shape=jax.ShapeDtypeStruct((B, 128), jnp.float32),
    )(avg, W, b2)

    out = pl.pallas_call(
        _normalize_body,
        grid=(NV,),
        in_specs=[
            pl.BlockSpec((B, D), lambda j: (0, 0)),
            pl.BlockSpec((D, VT), lambda j: (0, j)),
            pl.BlockSpec((1, VT), lambda j: (0, j)),
            pl.BlockSpec((B, 128), lambda j: (0, 0)),
        ],
        out_specs=pl.BlockSpec((B, VT), lambda j: (0, j)),
        out_shape=jax.ShapeDtypeStruct((B, V), jnp.float32),
    )(avg, W, b2, s128)
    return out


# speedup vs baseline: 1.4755x; 1.2659x over previous
import functools

import jax
import jax.numpy as jnp
from jax import lax
from jax.experimental import pallas as pl
from jax.experimental.pallas import tpu as pltpu
from jax.experimental.pallas import tpu_sc as plsc

V = 100000
D = 64
B = 1024
CTX = 200

NC = 2
NS = 16
NW = NC * NS
RPW = B // NW
GC0 = 104
GC1 = CTX - GC0

VT = 512
NV = (V + VT - 1) // VT
VP = NV * VT


def _gather_mean(inputs, table):
    mesh = plsc.VectorSubcoreMesh(
        core_axis_name="c", subcore_axis_name="s",
        num_cores=NC, num_subcores=NS)

    @functools.partial(
        pl.kernel,
        out_type=jax.ShapeDtypeStruct((B, D), jnp.float32),
        mesh=mesh,
        scratch_types=[
            pltpu.VMEM((RPW * CTX,), jnp.int32),
            pltpu.VMEM((CTX, D), jnp.float32),
            pltpu.VMEM((RPW, D), jnp.float32),
            pltpu.SemaphoreType.DMA,
        ],
        compiler_params=pltpu.CompilerParams(use_tc_tiling_on_sc=False),
    )
    def body(inputs_hbm, table_hbm, out_hbm, idx_v, rows_v, out_v, sem):
        wid = lax.axis_index("s") * NC + lax.axis_index("c")
        base = wid * RPW
        pltpu.sync_copy(inputs_hbm.at[pl.ds(base * CTX, RPW * CTX)], idx_v)

        def row_body(e, _):
            off = pl.multiple_of(e * CTX, 8)
            cp0 = pltpu.async_copy(
                table_hbm.at[idx_v.at[pl.ds(off, GC0)]],
                rows_v.at[pl.ds(0, GC0)], sem)
            cp1 = pltpu.async_copy(
                table_hbm.at[idx_v.at[pl.ds(pl.multiple_of(off + GC0, 8), GC1)]],
                rows_v.at[pl.ds(GC0, GC1)], sem)
            cp0.wait()
            cp1.wait()

            def acc_body(r, accs):
                return tuple(accs[g] + rows_v[r, pl.ds(g * 16, 16)]
                             for g in range(D // 16))

            accs = lax.fori_loop(
                0, CTX, acc_body,
                tuple(jnp.zeros((16,), jnp.float32) for _ in range(D // 16)))
            for g in range(D // 16):
                out_v[e, pl.ds(g * 16, 16)] = accs[g] * (1.0 / CTX)
            return 0

        lax.fori_loop(0, RPW, row_body, 0)
        pltpu.sync_copy(out_v, out_hbm.at[pl.ds(base, RPW)])

    return body(inputs.reshape(B * CTX), table)


def _sumexp_body(avg_ref, w_ref, b_ref, s_ref):
    j = pl.program_id(0)

    @pl.when(j == 0)
    def _():
        s_ref[...] = jnp.zeros_like(s_ref)

    logits = jnp.dot(avg_ref[...], w_ref[...],
                     preferred_element_type=jnp.float32) + b_ref[...]
    e = jnp.exp(logits)
    acc = s_ref[...]
    for k in range(VT // 128):
        acc = acc + e[:, k * 128:(k + 1) * 128]
    s_ref[...] = acc


def _normalize_body(avg_ref, w_ref, b_ref, s_ref, o_ref):
    logits = jnp.dot(avg_ref[...], w_ref[...],
                     preferred_element_type=jnp.float32) + b_ref[...]
    e = jnp.exp(logits)
    inv = 1.0 / jnp.sum(s_ref[...], axis=1, keepdims=True)
    o_ref[...] = e * inv


def kernel(inputs, table, W, b):
    avg = _gather_mean(inputs, table)
    Wp = jnp.pad(W, ((0, 0), (0, VP - V)))
    b2 = jnp.pad(b, (0, VP - V),
                 constant_values=-jnp.inf).reshape(1, VP)

    s128 = pl.pallas_call(
        _sumexp_body,
        grid=(NV,),
        in_specs=[
            pl.BlockSpec((B, D), lambda j: (0, 0)),
            pl.BlockSpec((D, VT), lambda j: (0, j)),
            pl.BlockSpec((1, VT), lambda j: (0, j)),
        ],
        out_specs=pl.BlockSpec((B, 128), lambda j: (0, 0)),
        out_shape=jax.ShapeDtypeStruct((B, 128), jnp.float32),
    )(avg, Wp, b2)

    out = pl.pallas_call(
        _normalize_body,
        grid=(NV,),
        in_specs=[
            pl.BlockSpec((B, D), lambda j: (0, 0)),
            pl.BlockSpec((D, VT), lambda j: (0, j)),
            pl.BlockSpec((1, VT), lambda j: (0, j)),
            pl.BlockSpec((B, 128), lambda j: (0, 0)),
        ],
        out_specs=pl.BlockSpec((B, VT), lambda j: (0, j)),
        out_shape=jax.ShapeDtypeStruct((B, V), jnp.float32),
    )(avg, Wp, b2, s128)
    return out


# speedup vs baseline: 2.4368x; 1.6515x over previous
import functools

import jax
import jax.numpy as jnp
from jax import lax
from jax.experimental import pallas as pl
from jax.experimental.pallas import tpu as pltpu
from jax.experimental.pallas import tpu_sc as plsc

V = 100000
D = 64
B = 1024
CTX = 200

NC = 2
NS = 16
NW = NC * NS
RPW = B // NW
GC0 = 104
GC1 = CTX - GC0

VT = 512
NV = (V + VT - 1) // VT
VP = NV * VT


def _gather_mean(inputs, table):
    mesh = plsc.VectorSubcoreMesh(
        core_axis_name="c", subcore_axis_name="s",
        num_cores=NC, num_subcores=NS)

    @functools.partial(
        pl.kernel,
        out_type=jax.ShapeDtypeStruct((B, D), jnp.float32),
        mesh=mesh,
        scratch_types=[
            pltpu.VMEM((RPW * CTX,), jnp.int32),
            pltpu.VMEM((CTX, D), jnp.float32),
            pltpu.VMEM((RPW, D), jnp.float32),
            pltpu.SemaphoreType.DMA,
        ],
        compiler_params=pltpu.CompilerParams(use_tc_tiling_on_sc=False),
    )
    def body(inputs_hbm, table_hbm, out_hbm, idx_v, rows_v, out_v, sem):
        wid = lax.axis_index("s") * NC + lax.axis_index("c")
        base = wid * RPW
        pltpu.sync_copy(inputs_hbm.at[pl.ds(base * CTX, RPW * CTX)], idx_v)

        def row_body(e, _):
            off = pl.multiple_of(e * CTX, 8)
            cp0 = pltpu.async_copy(
                table_hbm.at[idx_v.at[pl.ds(off, GC0)]],
                rows_v.at[pl.ds(0, GC0)], sem)
            cp1 = pltpu.async_copy(
                table_hbm.at[idx_v.at[pl.ds(pl.multiple_of(off + GC0, 8), GC1)]],
                rows_v.at[pl.ds(GC0, GC1)], sem)
            cp0.wait()
            cp1.wait()

            def acc_body(r, accs):
                return tuple(accs[g] + rows_v[r, pl.ds(g * 16, 16)]
                             for g in range(D // 16))

            accs = lax.fori_loop(
                0, CTX, acc_body,
                tuple(jnp.zeros((16,), jnp.float32) for _ in range(D // 16)))
            for g in range(D // 16):
                out_v[e, pl.ds(g * 16, 16)] = accs[g] * (1.0 / CTX)
            return 0

        lax.fori_loop(0, RPW, row_body, 0)
        pltpu.sync_copy(out_v, out_hbm.at[pl.ds(base, RPW)])

    return body(inputs.reshape(B * CTX), table)


_DN = (((0,), (0,)), ((), ()))


def _sumexp_body(a_ref, w_ref, s_ref):
    j = pl.program_id(0)

    @pl.when(j == 0)
    def _():
        s_ref[...] = jnp.zeros_like(s_ref)

    logits = lax.dot_general(w_ref[...], a_ref[...], _DN,
                             preferred_element_type=jnp.float32)
    e = jnp.exp(logits)
    ones = jnp.ones((1, VT), jnp.float32)
    part = jnp.dot(ones, e, preferred_element_type=jnp.float32)
    s_ref[...] += part


def _normalize_body(a_ref, w_ref, s_ref, o_ref):
    logits = lax.dot_general(w_ref[...], a_ref[...], _DN,
                             preferred_element_type=jnp.float32)
    e = jnp.exp(logits)
    inv = 1.0 / s_ref[...]
    o_ref[...] = e * inv


def kernel(inputs, table, W, b):
    avg = _gather_mean(inputs, table)
    aT = jnp.concatenate([avg.T, jnp.ones((1, B), jnp.float32)], axis=0)
    Wa = jnp.concatenate([
        jnp.pad(W, ((0, 0), (0, VP - V))),
        jnp.pad(b, (0, VP - V), constant_values=-jnp.inf).reshape(1, VP),
    ], axis=0)

    s = pl.pallas_call(
        _sumexp_body,
        grid=(NV,),
        in_specs=[
            pl.BlockSpec((D + 1, B), lambda j: (0, 0)),
            pl.BlockSpec((D + 1, VT), lambda j: (0, j)),
        ],
        out_specs=pl.BlockSpec((1, B), lambda j: (0, 0)),
        out_shape=jax.ShapeDtypeStruct((1, B), jnp.float32),
    )(aT, Wa)

    outT = pl.pallas_call(
        _normalize_body,
        grid=(NV,),
        in_specs=[
            pl.BlockSpec((D + 1, B), lambda j: (0, 0)),
            pl.BlockSpec((D + 1, VT), lambda j: (0, j)),
            pl.BlockSpec((1, B), lambda j: (0, 0)),
        ],
        out_specs=pl.BlockSpec((VT, B), lambda j: (j, 0)),
        out_shape=jax.ShapeDtypeStruct((V, B), jnp.float32),
    )(aT, Wa, s)
    return outT.T
